# Initial kernel scaffold; baseline (speedup 1.0000x reference)
import jax
import jax.numpy as jnp
from jax.experimental import pallas as pl


def kernel(x, codebook):
    raise NotImplementedError("write your pallas kernel here")


# speedup vs baseline: 1.8415x; 1.8415x over previous
import functools

import jax
import jax.numpy as jnp
from jax import lax
from jax.experimental import pallas as pl
from jax.experimental.pallas import tpu as pltpu
from jax.experimental.pallas import tpu_sc as plsc

G = 8
C = 128
K = 4096
NBLK = 256


def _round_body(refs, *, has_quant):
    i = 0
    x_ref = refs[i]; i += 1
    ps_ref = refs[i]; i += 1
    cb_ref = refs[i]; i += 1
    cbsq_ref = refs[i]; i += 1
    quant_ref = None
    if has_quant:
        quant_ref = refs[i]; i += 1
    idx_ref = refs[i]; i += 1

    t = x_ref[...] / ps_ref[...]
    if quant_ref is not None:
        t = t - quant_ref[...]

    cols = []
    for g in range(G):
        tg = t[:, g * C:(g + 1) * C]
        cb = cb_ref[g]
        dot = lax.dot_general(tg, cb, (((1,), (1,)), ((), ())),
                              preferred_element_type=jnp.float32)
        tsq = jnp.sum(tg * tg, axis=1, keepdims=True)
        d2 = tsq - 2.0 * dot + cbsq_ref[g]
        d2 = jnp.maximum(d2, 0.0)
        m = jnp.min(d2, axis=1, keepdims=True)
        iota = lax.broadcasted_iota(jnp.int32, (NBLK, K), 1)
        idxg = jnp.min(jnp.where(d2 == m, iota, K), axis=1)
        idxg = jnp.minimum(idxg, K - 1)
        cols.append(idxg.reshape(NBLK, 1))
    idx_ref[...] = jnp.concatenate(cols, axis=1)


def _round_call(xp, ps, cb, cbsq, quant):
    n = xp.shape[0]
    first = quant is None
    grid = (n // NBLK,)
    in_specs = [
        pl.BlockSpec((NBLK, G * C), lambda i: (i, 0)),
        pl.BlockSpec((NBLK, 1), lambda i: (i, 0)),
        pl.BlockSpec((G, K, C), lambda i: (0, 0, 0)),
        pl.BlockSpec((G, 1, K), lambda i: (0, 0, 0)),
    ]
    args = [xp, ps, cb, cbsq]
    if not first:
        in_specs.append(pl.BlockSpec((NBLK, G * C), lambda i: (i, 0)))
        args.append(quant)
    body = lambda *refs: _round_body(refs, has_quant=not first)
    return pl.pallas_call(
        body,
        grid=grid,
        in_specs=in_specs,
        out_specs=pl.BlockSpec((NBLK, G), lambda i: (i, 0)),
        out_shape=jax.ShapeDtypeStruct((n, G), jnp.int32),
        compiler_params=pltpu.CompilerParams(
            dimension_semantics=("arbitrary",)),
    )(*args)


def _gather_rows(table, idx_chunks, nw, b_per_w):
    nch = b_per_w // 128
    mesh = plsc.VectorSubcoreMesh(core_axis_name="c", subcore_axis_name="s")
    nc = mesh.num_cores

    @functools.partial(
        pl.kernel,
        out_type=jax.ShapeDtypeStruct((nw * b_per_w, C), jnp.float32),
        mesh=mesh,
        scratch_types=[
            pltpu.VMEM((nch, 128), jnp.int32),
            pltpu.VMEM((b_per_w, C), jnp.float32),
            pltpu.SemaphoreType.DMA,
        ],
    )
    def k(table_hbm, idx_hbm, out_hbm, idx_v, rows_v, sem):
        wid = lax.axis_index("s") * nc + lax.axis_index("c")
        pltpu.sync_copy(idx_hbm.at[wid], idx_v)
        descs = []
        for j in range(nch):
            descs.append(pltpu.async_copy(
                table_hbm.at[idx_v.at[j]],
                rows_v.at[pl.ds(j * 128, 128)], sem))
        for d in descs:
            d.wait()
        pltpu.sync_copy(rows_v, out_hbm.at[pl.ds(wid * b_per_w, b_per_w)])

    return k(table, idx_chunks)


def kernel(x, codebook):
    b, s, d = x.shape
    n = b * s
    xp = x.reshape(b, s, G, 2, 64).swapaxes(3, 4).reshape(n, G * C)
    ps = jnp.linalg.norm(xp, axis=-1, keepdims=True)
    cbsq = jnp.sum(codebook * codebook, axis=-1)
    cbsq = cbsq.reshape(codebook.shape[0], G, 1, K)

    idx1 = _round_call(xp, ps, codebook[0], cbsq[0], None)

    flat = (idx1 + (jnp.arange(G, dtype=jnp.int32) * K)[None, :]).reshape(-1)
    info = plsc.get_sparse_core_info()
    nw = info.num_cores * info.num_subcores
    b_per_w = (n * G) // nw
    quant = _gather_rows(codebook[0].reshape(G * K, C),
                         flat.reshape(nw, b_per_w // 128, 128), nw, b_per_w)

    idx2 = _round_call(xp, ps, codebook[1], cbsq[1], quant.reshape(n, G * C))

    codes = jnp.stack([idx1.T, idx2.T], axis=0).astype(jnp.uint16)
    prescale = ps.reshape(b, s, 1)
    return codes, prescale


# speedup vs baseline: 2.0435x; 1.1097x over previous
import functools

import jax
import jax.numpy as jnp
from jax import lax
from jax.experimental import pallas as pl
from jax.experimental.pallas import tpu as pltpu
from jax.experimental.pallas import tpu_sc as plsc

G = 8
C = 128
K = 4096
NBLK = 256


def _round_body(refs, *, has_quant):
    i = 0
    x_ref = refs[i]; i += 1
    ps_ref = refs[i]; i += 1
    cb_ref = refs[i]; i += 1
    cbsq_ref = refs[i]; i += 1
    quant_ref = None
    if has_quant:
        quant_ref = refs[i]; i += 1
    idx_ref = refs[i]; i += 1

    t = x_ref[...] / ps_ref[...]
    if quant_ref is not None:
        t = t - quant_ref[...]

    cols = []
    for g in range(G):
        tg = t[:, g * C:(g + 1) * C]
        cb = cb_ref[g]
        dot = lax.dot_general(tg, cb, (((1,), (1,)), ((), ())),
                              preferred_element_type=jnp.float32)
        tsq = jnp.sum(tg * tg, axis=1, keepdims=True)
        d2 = tsq - 2.0 * dot + cbsq_ref[g]
        d2 = jnp.maximum(d2, 0.0)
        idxg = jnp.argmin(d2, axis=1).astype(jnp.int32)
        cols.append(idxg.reshape(NBLK, 1))
    idx_ref[...] = jnp.concatenate(cols, axis=1)


def _round_call(xp, ps, cb, cbsq, quant):
    n = xp.shape[0]
    first = quant is None
    grid = (n // NBLK,)
    in_specs = [
        pl.BlockSpec((NBLK, G * C), lambda i: (i, 0)),
        pl.BlockSpec((NBLK, 1), lambda i: (i, 0)),
        pl.BlockSpec((G, K, C), lambda i: (0, 0, 0)),
        pl.BlockSpec((G, 1, K), lambda i: (0, 0, 0)),
    ]
    args = [xp, ps, cb, cbsq]
    if not first:
        in_specs.append(pl.BlockSpec((NBLK, G * C), lambda i: (i, 0)))
        args.append(quant)
    body = lambda *refs: _round_body(refs, has_quant=not first)
    return pl.pallas_call(
        body,
        grid=grid,
        in_specs=in_specs,
        out_specs=pl.BlockSpec((NBLK, G), lambda i: (i, 0)),
        out_shape=jax.ShapeDtypeStruct((n, G), jnp.int32),
        compiler_params=pltpu.CompilerParams(
            dimension_semantics=("arbitrary",)),
    )(*args)


def _gather_rows(table, idx_chunks, nw, b_per_w):
    nch = b_per_w // 128
    mesh = plsc.VectorSubcoreMesh(core_axis_name="c", subcore_axis_name="s")
    nc = mesh.num_cores

    @functools.partial(
        pl.kernel,
        out_type=jax.ShapeDtypeStruct((nw * b_per_w, C), jnp.float32),
        mesh=mesh,
        scratch_types=[
            pltpu.VMEM((nch, 128), jnp.int32),
            pltpu.VMEM((b_per_w, C), jnp.float32),
            pltpu.SemaphoreType.DMA,
        ],
    )
    def k(table_hbm, idx_hbm, out_hbm, idx_v, rows_v, sem):
        wid = lax.axis_index("s") * nc + lax.axis_index("c")
        pltpu.sync_copy(idx_hbm.at[wid], idx_v)
        descs = []
        for j in range(nch):
            descs.append(pltpu.async_copy(
                table_hbm.at[idx_v.at[j]],
                rows_v.at[pl.ds(j * 128, 128)], sem))
        for d in descs:
            d.wait()
        pltpu.sync_copy(rows_v, out_hbm.at[pl.ds(wid * b_per_w, b_per_w)])

    return k(table, idx_chunks)


def kernel(x, codebook):
    b, s, d = x.shape
    n = b * s
    xp = x.reshape(b, s, G, 2, 64).swapaxes(3, 4).reshape(n, G * C)
    ps = jnp.linalg.norm(xp, axis=-1, keepdims=True)
    cbsq = jnp.sum(codebook * codebook, axis=-1)
    cbsq = cbsq.reshape(codebook.shape[0], G, 1, K)

    idx1 = _round_call(xp, ps, codebook[0], cbsq[0], None)

    flat = (idx1 + (jnp.arange(G, dtype=jnp.int32) * K)[None, :]).reshape(-1)
    info = plsc.get_sparse_core_info()
    nw = info.num_cores * info.num_subcores
    b_per_w = (n * G) // nw
    quant = _gather_rows(codebook[0].reshape(G * K, C),
                         flat.reshape(nw, b_per_w // 128, 128), nw, b_per_w)

    idx2 = _round_call(xp, ps, codebook[1], cbsq[1], quant.reshape(n, G * C))

    codes = jnp.stack([idx1.T, idx2.T], axis=0).astype(jnp.uint16)
    prescale = ps.reshape(b, s, 1)
    return codes, prescale


# speedup vs baseline: 2.8843x; 1.4114x over previous
import functools

import jax
import jax.numpy as jnp
from jax import lax
from jax.experimental import pallas as pl
from jax.experimental.pallas import tpu as pltpu
from jax.experimental.pallas import tpu_sc as plsc

G = 8
C = 128
K = 4096
NBLK = 256


def _round_body(refs, *, has_quant):
    i = 0
    x_ref = refs[i]; i += 1
    ps_ref = refs[i]; i += 1
    cb_ref = refs[i]; i += 1
    cbsq_ref = refs[i]; i += 1
    quant_ref = None
    if has_quant:
        quant_ref = refs[i]; i += 1
    idx_ref = refs[i]; i += 1

    ts = x_ref[...] / ps_ref[...]
    if quant_ref is not None:
        ts = ts + 2.0 * quant_ref[...]

    cols = []
    for g in range(G):
        tg = ts[:, g * C:(g + 1) * C]
        cb = cb_ref[g]
        dot = lax.dot_general(tg, cb, (((1,), (1,)), ((), ())),
                              preferred_element_type=jnp.float32)
        score = dot + cbsq_ref[g]
        idxg = jnp.argmin(score, axis=1).astype(jnp.int32)
        cols.append(idxg.reshape(NBLK, 1))
    idx_ref[...] = jnp.concatenate(cols, axis=1)


def _round_call(xp, ps, cb, cbsq, quant):
    n = xp.shape[0]
    first = quant is None
    grid = (n // NBLK,)
    in_specs = [
        pl.BlockSpec((NBLK, G * C), lambda i: (i, 0)),
        pl.BlockSpec((NBLK, 1), lambda i: (i, 0)),
        pl.BlockSpec((G, K, C), lambda i: (0, 0, 0)),
        pl.BlockSpec((G, 1, K), lambda i: (0, 0, 0)),
    ]
    args = [xp, ps, cb, cbsq]
    if not first:
        in_specs.append(pl.BlockSpec((NBLK, G * C), lambda i: (i, 0)))
        args.append(quant)
    body = lambda *refs: _round_body(refs, has_quant=not first)
    return pl.pallas_call(
        body,
        grid=grid,
        in_specs=in_specs,
        out_specs=pl.BlockSpec((NBLK, G), lambda i: (i, 0)),
        out_shape=jax.ShapeDtypeStruct((n, G), jnp.int32),
        compiler_params=pltpu.CompilerParams(
            dimension_semantics=("arbitrary",)),
    )(*args)


def _gather_rows(table, idx_chunks, nw, b_per_w):
    nch = b_per_w // 128
    mesh = plsc.VectorSubcoreMesh(core_axis_name="c", subcore_axis_name="s")
    nc = mesh.num_cores

    @functools.partial(
        pl.kernel,
        out_type=jax.ShapeDtypeStruct((nw * b_per_w, C), jnp.float32),
        mesh=mesh,
        scratch_types=[
            pltpu.VMEM((nch, 128), jnp.int32),
            pltpu.VMEM((b_per_w, C), jnp.float32),
            pltpu.SemaphoreType.DMA,
        ],
    )
    def k(table_hbm, idx_hbm, out_hbm, idx_v, rows_v, sem):
        wid = lax.axis_index("s") * nc + lax.axis_index("c")
        pltpu.sync_copy(idx_hbm.at[wid], idx_v)
        descs = []
        for j in range(nch):
            descs.append(pltpu.async_copy(
                table_hbm.at[idx_v.at[j]],
                rows_v.at[pl.ds(j * 128, 128)], sem))
        for d in descs:
            d.wait()
        pltpu.sync_copy(rows_v, out_hbm.at[pl.ds(wid * b_per_w, b_per_w)])

    return k(table, idx_chunks)


def kernel(x, codebook):
    b, s, d = x.shape
    n = b * s
    xp = x.reshape(b, s, G, 2, 64).swapaxes(3, 4).reshape(n, G * C)
    ps = jnp.linalg.norm(xp, axis=-1, keepdims=True)
    psm = ps * -0.5
    cbsq = jnp.sum(codebook * codebook, axis=-1)
    cbsq = cbsq.reshape(codebook.shape[0], G, 1, K)

    idx1 = _round_call(xp, psm, codebook[0], cbsq[0], None)

    flat = (idx1 + (jnp.arange(G, dtype=jnp.int32) * K)[None, :]).reshape(-1)
    info = plsc.get_sparse_core_info()
    nw = info.num_cores * info.num_subcores
    b_per_w = (n * G) // nw
    quant = _gather_rows(codebook[0].reshape(G * K, C),
                         flat.reshape(nw, b_per_w // 128, 128), nw, b_per_w)

    idx2 = _round_call(xp, psm, codebook[1], cbsq[1], quant.reshape(n, G * C))

    codes = jnp.stack([idx1.T, idx2.T], axis=0).astype(jnp.uint16)
    prescale = ps.reshape(b, s, 1)
    return codes, prescale
